# baseline (speedup 1.0000x reference)
import functools
import jax
import jax.numpy as jnp
from jax.experimental import pallas as pl

DIM = 128
ALPHA = 0.2
NUM_NODE = 10000
N_ITEM = 10000
N_TARGET = 1024


def _norm_body(f_ref, s_ref, o_ref):
    f = f_ref[0] + f_ref[1]
    s = jnp.sum(s_ref[...], axis=1) + 1e-9
    o_ref[...] = f / s[:, None]


def _normalize(ftnum2, s_parts_t):
    R = 1000
    grid = (N_ITEM // R,)
    return pl.pallas_call(
        _norm_body,
        grid=grid,
        in_specs=[
            pl.BlockSpec((2, R, DIM), lambda i: (0, i, 0)),
            pl.BlockSpec((R, s_parts_t.shape[1]), lambda i: (i, 0)),
        ],
        out_specs=pl.BlockSpec((R, DIM), lambda i: (i, 0)),
        out_shape=jax.ShapeDtypeStruct((N_ITEM, DIM), jnp.float32),
    )(ftnum2, s_parts_t)


def _s2_body(fte_ref, hp_ref, qa_ref, qb_ref, t0_ref, o_ref):
    z = jnp.dot(fte_ref[...], qa_ref[...], preferred_element_type=jnp.float32)
    z = z + jnp.dot(hp_ref[...], qb_ref[...], preferred_element_type=jnp.float32)
    s2 = jnp.sum(jnp.tanh(z) * t0_ref[...], axis=-1)
    o_ref[...] = s2.reshape(o_ref.shape)


def _s2_compute(fte, hp, qaT, qbT, t0):
    npad = fte.shape[0]
    nblk = npad // 1024
    out = pl.pallas_call(
        _s2_body,
        grid=(nblk,),
        in_specs=[
            pl.BlockSpec((1024, DIM), lambda i: (i, 0)),
            pl.BlockSpec((1024, DIM), lambda i: (i, 0)),
            pl.BlockSpec((DIM, DIM), lambda i: (0, 0)),
            pl.BlockSpec((DIM, DIM), lambda i: (0, 0)),
            pl.BlockSpec((1, DIM), lambda i: (0, 0)),
        ],
        out_specs=pl.BlockSpec((8, 128), lambda i: (i, 0)),
        out_shape=jax.ShapeDtypeStruct((nblk * 8, 128), jnp.float32),
    )(fte, hp, qaT, qbT, t0)
    return out.reshape(npad)


def _score_body(sel_ref, emb_ref, o_ref):
    o_ref[...] = jax.lax.dot_general(
        sel_ref[...], emb_ref[...],
        dimension_numbers=(((1,), (1,)), ((), ())),
        preferred_element_type=jnp.float32)


def _scores(select, emb_pad):
    NBLK = 2048
    npad = emb_pad.shape[0]
    grid = (npad // NBLK,)
    return pl.pallas_call(
        _score_body,
        grid=grid,
        in_specs=[
            pl.BlockSpec((N_TARGET, DIM), lambda i: (0, 0)),
            pl.BlockSpec((NBLK, DIM), lambda i: (i, 0)),
        ],
        out_specs=pl.BlockSpec((N_TARGET, NBLK), lambda i: (0, i)),
        out_shape=jax.ShapeDtypeStruct((N_TARGET, npad), jnp.float32),
    )(select, emb_pad)


def kernel(item_ids, edge_index, pid, tid, agg_src, agg_dst,
           embedding, pos_embedding, target_embedding, p_w, q_w):
    src = edge_index[0]
    dst = edge_index[1]

    gsrc = item_ids[src]
    gdst = item_ids[dst]
    ft_src = embedding[gsrc]
    ft_dst = embedding[gdst]
    e = (ft_src * ft_dst) @ p_w
    e = jnp.maximum(e, ALPHA * e)
    w = jnp.exp(e)
    s = jax.ops.segment_sum(w, dst, num_segments=N_ITEM)
    ftnum = jax.ops.segment_sum(ft_src * w[:, None], dst, num_segments=N_ITEM)
    ftnum2 = jnp.stack([ftnum, jnp.zeros_like(ftnum)])
    s_parts_t = s[:, None]

    ft = _normalize(ftnum2, s_parts_t)

    E_AGG = agg_src.shape[0]
    npad = ((E_AGG + 1023) // 1024) * 1024
    fte = ft[agg_src]
    hp = pos_embedding[pid]
    fte_p = jnp.pad(fte, ((0, npad - E_AGG), (0, 0)))
    hp_p = jnp.pad(hp, ((0, npad - E_AGG), (0, 0)))
    qaT = q_w[:, :DIM].T
    qbT = q_w[:, DIM:].T
    t0 = target_embedding[0:1]

    s2 = _s2_compute(fte_p, hp_p, qaT, qbT, t0)[:E_AGG]

    m = fte * s2[:, None]
    select = jax.ops.segment_sum(m, agg_dst, num_segments=N_TARGET)

    emb_pad = jnp.pad(embedding, ((0, 10240 - NUM_NODE), (0, 0)))
    full = _scores(select, emb_pad)
    return full[:, 1:NUM_NODE]


# speedup vs baseline: 2.0469x; 2.0469x over previous
import functools
import jax
import jax.numpy as jnp
from jax import lax
from jax.experimental import pallas as pl
from jax.experimental.pallas import tpu as pltpu
from jax.experimental.pallas import tpu_sc as plsc

DIM = 128
ALPHA = 0.2
NUM_NODE = 10000
N_ITEM = 10000
N_TARGET = 1024
E_INT = 320000
E_AGG = 50000

NW = 32
EPW = E_INT // NW
C2 = 80
NCH2 = EPW // C2

EA_PAD = 50176
EAPW = EA_PAD // NW
C4 = 112
NCH4 = EAPW // C4

ROWS_PER_TILE = 624
SEL_PER_TILE = N_TARGET // 16

_mesh = plsc.VectorSubcoreMesh(core_axis_name="c", subcore_axis_name="s")
_sc_params = pltpu.CompilerParams(needs_layout_passes=False)


def _zero_vmem_2d(ref, nrows):
    z = jnp.zeros((16,), jnp.float32)
    for r in range(nrows):
        for j in range(DIM // 16):
            ref[r, pl.ds(16 * j, 16)] = z


def _edge_body(src_hbm, dst_hbm, iid_hbm, emb_hbm, embp_hbm,
               ftnum_hbm, s_hbm,
               iid_v, src_v, dst_v, gs_v, gd_v,
               rows_s, rows_d, zb_v, s_hist, ft_sh, sem1, sem2):
    cid = lax.axis_index("c")
    sid = lax.axis_index("s")
    wid = sid * 2 + cid

    pltpu.sync_copy(iid_hbm, iid_v)

    zf = jnp.zeros((16,), jnp.float32)
    for i in range(N_ITEM // 16):
        s_hist[pl.ds(16 * i, 16)] = zf
    _zero_vmem_2d(zb_v, 48)
    for t in range(ROWS_PER_TILE // 48):
        pltpu.sync_copy(zb_v, ft_sh.at[pl.ds(sid * ROWS_PER_TILE + t * 48, 48)])

    @pl.when(sid == 15)
    def _():
        pltpu.sync_copy(zb_v.at[pl.ds(0, 16)], ft_sh.at[pl.ds(9984, 16)])

    plsc.subcore_barrier()

    iota16 = lax.iota(jnp.int32, 16)
    one_i = jnp.ones((16,), jnp.int32)
    zero_i = jnp.zeros((16,), jnp.int32)

    def chunk(i, carry):
        base = wid * EPW + i * C2
        pltpu.sync_copy(src_hbm.at[pl.ds(base, C2)], src_v)
        pltpu.sync_copy(dst_hbm.at[pl.ds(base, C2)], dst_v)
        for j in range(C2 // 16):
            sl = pl.ds(16 * j, 16)
            gs_v[sl] = plsc.load_gather(iid_v, [src_v[sl]])
            gd_v[sl] = plsc.load_gather(iid_v, [dst_v[sl]])
        cp1 = pltpu.async_copy(emb_hbm.at[gs_v], rows_s, sem1)
        cp2 = pltpu.async_copy(embp_hbm.at[gd_v], rows_d, sem2)
        cp1.wait()
        cp2.wait()
        for g in range(C2 // 16):
            ridx = iota16 + (16 * g)

            def dot_step(d, c):
                acc, cidx = c
                vs = plsc.load_gather(rows_s, [ridx, cidx])
                vd = plsc.load_gather(rows_d, [ridx, cidx])
                return acc + vs * vd, cidx + one_i

            acc, _ = lax.fori_loop(0, DIM, dot_step, (zf, zero_i), unroll=8)
            ev = jnp.maximum(acc, ALPHA * acc)
            wv = jnp.exp(ev)
            dst16 = dst_v[pl.ds(16 * g, 16)]
            plsc.addupdate_scatter(s_hist, [dst16], wv)

            def scale_step(d, cidx):
                vs = plsc.load_gather(rows_s, [ridx, cidx])
                plsc.store_scatter(rows_s, [ridx, cidx], vs * wv)
                return cidx + one_i

            lax.fori_loop(0, DIM, scale_step, zero_i, unroll=8)
        pltpu.sync_copy(rows_s, ft_sh.at[dst_v], add=True)
        return carry

    lax.fori_loop(0, NCH2, chunk, 0)
    plsc.subcore_barrier()

    r0 = sid * ROWS_PER_TILE
    pltpu.sync_copy(ft_sh.at[pl.ds(r0, ROWS_PER_TILE)],
                    ftnum_hbm.at[pl.ds(cid * N_ITEM + r0, ROWS_PER_TILE)])

    @pl.when(sid == 15)
    def _():
        pltpu.sync_copy(ft_sh.at[pl.ds(9984, 16)],
                        ftnum_hbm.at[pl.ds(cid * N_ITEM + 9984, 16)])

    pltpu.sync_copy(s_hist, s_hbm.at[pl.ds(wid * N_ITEM, N_ITEM)])


_edge_call = pl.kernel(
    _edge_body,
    out_type=(jax.ShapeDtypeStruct((2 * N_ITEM, DIM), jnp.float32),
              jax.ShapeDtypeStruct((NW * N_ITEM,), jnp.float32)),
    mesh=_mesh,
    compiler_params=_sc_params,
    scratch_types=[
        pltpu.VMEM((N_ITEM,), jnp.int32),
        pltpu.VMEM((C2,), jnp.int32),
        pltpu.VMEM((C2,), jnp.int32),
        pltpu.VMEM((C2,), jnp.int32),
        pltpu.VMEM((C2,), jnp.int32),
        pltpu.VMEM((C2, DIM), jnp.float32),
        pltpu.VMEM((C2, DIM), jnp.float32),
        pltpu.VMEM((48, DIM), jnp.float32),
        pltpu.VMEM((N_ITEM,), jnp.float32),
        pltpu.VMEM_SHARED((N_ITEM, DIM), jnp.float32),
        pltpu.SemaphoreType.DMA,
        pltpu.SemaphoreType.DMA,
    ],
)


def _gather_body(asrc_hbm, pid_hbm, ft_hbm, pos_hbm,
                 fte_hbm, hp_hbm,
                 idx_v, idx2_v, buf1, buf2, sem1, sem2):
    cid = lax.axis_index("c")
    sid = lax.axis_index("s")
    wid = sid * 2 + cid

    def chunk(i, carry):
        base = wid * EAPW + i * C4
        pltpu.sync_copy(asrc_hbm.at[pl.ds(base, C4)], idx_v)
        pltpu.sync_copy(pid_hbm.at[pl.ds(base, C4)], idx2_v)
        cp1 = pltpu.async_copy(ft_hbm.at[idx_v], buf1, sem1)
        cp2 = pltpu.async_copy(pos_hbm.at[idx2_v], buf2, sem2)
        cp1.wait()
        cp2.wait()
        pltpu.sync_copy(buf1, fte_hbm.at[pl.ds(base, C4)])
        pltpu.sync_copy(buf2, hp_hbm.at[pl.ds(base, C4)])
        return carry

    lax.fori_loop(0, NCH4, chunk, 0)


_gather_call = pl.kernel(
    _gather_body,
    out_type=(jax.ShapeDtypeStruct((EA_PAD, DIM), jnp.float32),
              jax.ShapeDtypeStruct((EA_PAD, DIM), jnp.float32)),
    mesh=_mesh,
    compiler_params=_sc_params,
    scratch_types=[
        pltpu.VMEM((C4,), jnp.int32),
        pltpu.VMEM((C4,), jnp.int32),
        pltpu.VMEM((C4, DIM), jnp.float32),
        pltpu.VMEM((C4, DIM), jnp.float32),
        pltpu.SemaphoreType.DMA,
        pltpu.SemaphoreType.DMA,
    ],
)


def _select_body(fte_hbm, s2_hbm, adst_hbm,
                 sel_hbm,
                 dst_v, s2_v, rows_v, zb_v, sel_sh, sem1):
    cid = lax.axis_index("c")
    sid = lax.axis_index("s")
    wid = sid * 2 + cid

    _zero_vmem_2d(zb_v, 16)
    for t in range(SEL_PER_TILE // 16):
        pltpu.sync_copy(zb_v, sel_sh.at[pl.ds(sid * SEL_PER_TILE + t * 16, 16)])
    plsc.subcore_barrier()

    iota16 = lax.iota(jnp.int32, 16)
    one_i = jnp.ones((16,), jnp.int32)
    zero_i = jnp.zeros((16,), jnp.int32)

    def chunk(i, carry):
        base = wid * EAPW + i * C4
        cp1 = pltpu.async_copy(fte_hbm.at[pl.ds(base, C4)], rows_v, sem1)
        pltpu.sync_copy(s2_hbm.at[pl.ds(base, C4)], s2_v)
        pltpu.sync_copy(adst_hbm.at[pl.ds(base, C4)], dst_v)
        cp1.wait()
        for g in range(C4 // 16):
            ridx = iota16 + (16 * g)
            wv = s2_v[pl.ds(16 * g, 16)]

            def scale_step(d, cidx):
                vs = plsc.load_gather(rows_v, [ridx, cidx])
                plsc.store_scatter(rows_v, [ridx, cidx], vs * wv)
                return cidx + one_i

            lax.fori_loop(0, DIM, scale_step, zero_i, unroll=8)
        pltpu.sync_copy(rows_v, sel_sh.at[dst_v], add=True)
        return carry

    lax.fori_loop(0, NCH4, chunk, 0)
    plsc.subcore_barrier()

    r0 = sid * SEL_PER_TILE
    pltpu.sync_copy(sel_sh.at[pl.ds(r0, SEL_PER_TILE)],
                    sel_hbm.at[pl.ds(cid * N_TARGET + r0, SEL_PER_TILE)])


_select_call = pl.kernel(
    _select_body,
    out_type=jax.ShapeDtypeStruct((2 * N_TARGET, DIM), jnp.float32),
    mesh=_mesh,
    compiler_params=_sc_params,
    scratch_types=[
        pltpu.VMEM((C4,), jnp.int32),
        pltpu.VMEM((C4,), jnp.float32),
        pltpu.VMEM((C4, DIM), jnp.float32),
        pltpu.VMEM((16, DIM), jnp.float32),
        pltpu.VMEM_SHARED((N_TARGET, DIM), jnp.float32),
        pltpu.SemaphoreType.DMA,
    ],
)


def _premul_body(e_ref, pw_ref, o_ref):
    o_ref[...] = e_ref[...] * pw_ref[...]


def _premul(embedding, p_w_row):
    R = 2000
    return pl.pallas_call(
        _premul_body,
        grid=(N_ITEM // R,),
        in_specs=[
            pl.BlockSpec((R, DIM), lambda i: (i, 0)),
            pl.BlockSpec((1, DIM), lambda i: (0, 0)),
        ],
        out_specs=pl.BlockSpec((R, DIM), lambda i: (i, 0)),
        out_shape=jax.ShapeDtypeStruct((N_ITEM, DIM), jnp.float32),
    )(embedding, p_w_row)


def _norm_body(f_ref, s_ref, o_ref):
    f = f_ref[0] + f_ref[1]
    s = jnp.sum(s_ref[...], axis=1) + 1e-9
    o_ref[...] = f / s[:, None]


def _normalize(ftnum2, s_parts_t):
    R = 1000
    return pl.pallas_call(
        _norm_body,
        grid=(N_ITEM // R,),
        in_specs=[
            pl.BlockSpec((2, R, DIM), lambda i: (0, i, 0)),
            pl.BlockSpec((R, s_parts_t.shape[1]), lambda i: (i, 0)),
        ],
        out_specs=pl.BlockSpec((R, DIM), lambda i: (i, 0)),
        out_shape=jax.ShapeDtypeStruct((N_ITEM, DIM), jnp.float32),
    )(ftnum2, s_parts_t)


def _s2_body(fte_ref, hp_ref, qa_ref, qb_ref, t0_ref, o_ref):
    i = pl.program_id(0)
    z = jnp.dot(fte_ref[...], qa_ref[...], preferred_element_type=jnp.float32)
    z = z + jnp.dot(hp_ref[...], qb_ref[...], preferred_element_type=jnp.float32)
    s2 = jnp.sum(jnp.tanh(z) * t0_ref[...], axis=-1).reshape(o_ref.shape)
    rid = (i * 1024
           + lax.broadcasted_iota(jnp.int32, o_ref.shape, 0) * 128
           + lax.broadcasted_iota(jnp.int32, o_ref.shape, 1))
    o_ref[...] = jnp.where(rid < E_AGG, s2, 0.0)


def _s2_compute(fte, hp, qaT, qbT, t0):
    nblk = EA_PAD // 1024
    out = pl.pallas_call(
        _s2_body,
        grid=(nblk,),
        in_specs=[
            pl.BlockSpec((1024, DIM), lambda i: (i, 0)),
            pl.BlockSpec((1024, DIM), lambda i: (i, 0)),
            pl.BlockSpec((DIM, DIM), lambda i: (0, 0)),
            pl.BlockSpec((DIM, DIM), lambda i: (0, 0)),
            pl.BlockSpec((1, DIM), lambda i: (0, 0)),
        ],
        out_specs=pl.BlockSpec((8, 128), lambda i: (i, 0)),
        out_shape=jax.ShapeDtypeStruct((nblk * 8, 128), jnp.float32),
    )(fte, hp, qaT, qbT, t0)
    return out.reshape(EA_PAD)


def _score_body(sel_ref, emb_ref, o_ref):
    sel = sel_ref[0] + sel_ref[1]
    o_ref[...] = lax.dot_general(
        sel, emb_ref[...],
        dimension_numbers=(((1,), (1,)), ((), ())),
        preferred_element_type=jnp.float32)


def _scores(sel2, emb_pad):
    NBLK = 2048
    npad = emb_pad.shape[0]
    return pl.pallas_call(
        _score_body,
        grid=(npad // NBLK,),
        in_specs=[
            pl.BlockSpec((2, N_TARGET, DIM), lambda i: (0, 0, 0)),
            pl.BlockSpec((NBLK, DIM), lambda i: (i, 0)),
        ],
        out_specs=pl.BlockSpec((N_TARGET, NBLK), lambda i: (0, i)),
        out_shape=jax.ShapeDtypeStruct((N_TARGET, npad), jnp.float32),
    )(sel2, emb_pad)


def kernel(item_ids, edge_index, pid, tid, agg_src, agg_dst,
           embedding, pos_embedding, target_embedding, p_w, q_w):
    src = edge_index[0]
    dst = edge_index[1]

    embp = _premul(embedding, p_w[None, :])
    ftnum_flat, s_parts = _edge_call(src, dst, item_ids, embedding, embp)
    ftnum2 = ftnum_flat.reshape(2, N_ITEM, DIM)
    s_parts = s_parts.reshape(NW, N_ITEM)
    ft = _normalize(ftnum2, s_parts.T)

    pad_a = EA_PAD - E_AGG
    asrc_p = jnp.pad(agg_src, (0, pad_a))
    adst_p = jnp.pad(agg_dst, (0, pad_a))
    pid_p = jnp.pad(pid, (0, pad_a))

    fte, hp = _gather_call(asrc_p, pid_p, ft, pos_embedding)

    qaT = q_w[:, :DIM].T
    qbT = q_w[:, DIM:].T
    t0 = target_embedding[0:1]
    s2 = _s2_compute(fte, hp, qaT, qbT, t0)

    sel_flat = _select_call(fte, s2, adst_p)
    sel2 = sel_flat.reshape(2, N_TARGET, DIM)

    emb_pad = jnp.pad(embedding, ((0, 10240 - NUM_NODE), (0, 0)))
    full = _scores(sel2, emb_pad)
    return full[:, 1:NUM_NODE]


# speedup vs baseline: 6.1464x; 3.0028x over previous
import functools
import jax
import jax.numpy as jnp
from jax import lax
from jax.experimental import pallas as pl
from jax.experimental.pallas import tpu as pltpu
from jax.experimental.pallas import tpu_sc as plsc

DIM = 128
ALPHA = 0.2
NUM_NODE = 10000
N_ITEM = 10000
N_TARGET = 1024
E_INT = 320000
E_AGG = 50000

NW = 32
EPW = E_INT // NW
C2 = 80
NCH2 = EPW // C2

EA_PAD = 50176
EAPW = EA_PAD // NW
C4 = 112
NCH4 = EAPW // C4

ROWS_PER_TILE = 624
SEL_PER_TILE = N_TARGET // 16

_mesh = plsc.VectorSubcoreMesh(core_axis_name="c", subcore_axis_name="s")
_sc_params = pltpu.CompilerParams(needs_layout_passes=False)


def _zero_vmem_2d(ref, nrows):
    z = jnp.zeros((16,), jnp.float32)
    for r in range(nrows):
        for j in range(DIM // 16):
            ref[r, pl.ds(16 * j, 16)] = z


def _edge_body(src_hbm, dst_hbm, iid_hbm, emb_hbm, embp_hbm,
               ftnum_hbm, s_hbm,
               iid_v, src_v, dst_v, gs_v, gd_v,
               rows_s, rows_d, zb_v, s_hist, ft_sh, sem1, sem2):
    cid = lax.axis_index("c")
    sid = lax.axis_index("s")
    wid = sid * 2 + cid

    pltpu.sync_copy(iid_hbm, iid_v)

    zf = jnp.zeros((16,), jnp.float32)
    for i in range(N_ITEM // 16):
        s_hist[pl.ds(16 * i, 16)] = zf
    _zero_vmem_2d(zb_v, 48)
    for t in range(ROWS_PER_TILE // 48):
        pltpu.sync_copy(zb_v, ft_sh.at[pl.ds(sid * ROWS_PER_TILE + t * 48, 48)])

    @pl.when(sid == 15)
    def _():
        pltpu.sync_copy(zb_v.at[pl.ds(0, 16)], ft_sh.at[pl.ds(9984, 16)])

    plsc.subcore_barrier()

    iota16 = lax.iota(jnp.int32, 16)
    one_i = jnp.ones((16,), jnp.int32)
    zero_i = jnp.zeros((16,), jnp.int32)

    def chunk(i, carry):
        base = wid * EPW + i * C2
        pltpu.sync_copy(src_hbm.at[pl.ds(base, C2)], src_v)
        pltpu.sync_copy(dst_hbm.at[pl.ds(base, C2)], dst_v)
        for j in range(C2 // 16):
            sl = pl.ds(16 * j, 16)
            gs_v[sl] = plsc.load_gather(iid_v, [src_v[sl]])
            gd_v[sl] = plsc.load_gather(iid_v, [dst_v[sl]])
        cp1 = pltpu.async_copy(emb_hbm.at[gs_v], rows_s, sem1)
        cp2 = pltpu.async_copy(embp_hbm.at[gd_v], rows_d, sem2)
        cp1.wait()
        cp2.wait()
        for g in range(C2 // 16):
            ridx = iota16 + (16 * g)

            def dot_step(d, c):
                acc, cidx = c
                vs = plsc.load_gather(rows_s, [ridx, cidx])
                vd = plsc.load_gather(rows_d, [ridx, cidx])
                return acc + vs * vd, (cidx + one_i) & 127

            acc, _ = lax.fori_loop(0, DIM, dot_step, (zf, iota16), unroll=8)
            ev = jnp.maximum(acc, ALPHA * acc)
            wv = jnp.exp(ev)
            dst16 = dst_v[pl.ds(16 * g, 16)]
            plsc.addupdate_scatter(s_hist, [dst16], wv)

            def scale_step(d, cidx):
                vs = plsc.load_gather(rows_s, [ridx, cidx])
                plsc.store_scatter(rows_s, [ridx, cidx], vs * wv)
                return (cidx + one_i) & 127

            lax.fori_loop(0, DIM, scale_step, iota16, unroll=8)
        pltpu.sync_copy(rows_s, ft_sh.at[dst_v], add=True)
        return carry

    lax.fori_loop(0, NCH2, chunk, 0)
    plsc.subcore_barrier()

    r0 = sid * ROWS_PER_TILE
    pltpu.sync_copy(ft_sh.at[pl.ds(r0, ROWS_PER_TILE)],
                    ftnum_hbm.at[pl.ds(cid * N_ITEM + r0, ROWS_PER_TILE)])

    @pl.when(sid == 15)
    def _():
        pltpu.sync_copy(ft_sh.at[pl.ds(9984, 16)],
                        ftnum_hbm.at[pl.ds(cid * N_ITEM + 9984, 16)])

    pltpu.sync_copy(s_hist, s_hbm.at[pl.ds(wid * N_ITEM, N_ITEM)])


_edge_call = pl.kernel(
    _edge_body,
    out_type=(jax.ShapeDtypeStruct((2 * N_ITEM, DIM), jnp.float32),
              jax.ShapeDtypeStruct((NW * N_ITEM,), jnp.float32)),
    mesh=_mesh,
    compiler_params=_sc_params,
    scratch_types=[
        pltpu.VMEM((N_ITEM,), jnp.int32),
        pltpu.VMEM((C2,), jnp.int32),
        pltpu.VMEM((C2,), jnp.int32),
        pltpu.VMEM((C2,), jnp.int32),
        pltpu.VMEM((C2,), jnp.int32),
        pltpu.VMEM((C2, DIM), jnp.float32),
        pltpu.VMEM((C2, DIM), jnp.float32),
        pltpu.VMEM((48, DIM), jnp.float32),
        pltpu.VMEM((N_ITEM,), jnp.float32),
        pltpu.VMEM_SHARED((N_ITEM, DIM), jnp.float32),
        pltpu.SemaphoreType.DMA,
        pltpu.SemaphoreType.DMA,
    ],
)


def _gather_body(asrc_hbm, pid_hbm, ft_hbm, pos_hbm,
                 fte_hbm, hp_hbm,
                 idx_v, idx2_v, buf1, buf2, sem1, sem2):
    cid = lax.axis_index("c")
    sid = lax.axis_index("s")
    wid = sid * 2 + cid

    def chunk(i, carry):
        base = wid * EAPW + i * C4
        pltpu.sync_copy(asrc_hbm.at[pl.ds(base, C4)], idx_v)
        pltpu.sync_copy(pid_hbm.at[pl.ds(base, C4)], idx2_v)
        cp1 = pltpu.async_copy(ft_hbm.at[idx_v], buf1, sem1)
        cp2 = pltpu.async_copy(pos_hbm.at[idx2_v], buf2, sem2)
        cp1.wait()
        cp2.wait()
        pltpu.sync_copy(buf1, fte_hbm.at[pl.ds(base, C4)])
        pltpu.sync_copy(buf2, hp_hbm.at[pl.ds(base, C4)])
        return carry

    lax.fori_loop(0, NCH4, chunk, 0)


_gather_call = pl.kernel(
    _gather_body,
    out_type=(jax.ShapeDtypeStruct((EA_PAD, DIM), jnp.float32),
              jax.ShapeDtypeStruct((EA_PAD, DIM), jnp.float32)),
    mesh=_mesh,
    compiler_params=_sc_params,
    scratch_types=[
        pltpu.VMEM((C4,), jnp.int32),
        pltpu.VMEM((C4,), jnp.int32),
        pltpu.VMEM((C4, DIM), jnp.float32),
        pltpu.VMEM((C4, DIM), jnp.float32),
        pltpu.SemaphoreType.DMA,
        pltpu.SemaphoreType.DMA,
    ],
)


def _select_body(fte_hbm, s2_hbm, adst_hbm,
                 sel_hbm,
                 dst_v, s2_v, rows_v, zb_v, sel_sh, sem1):
    cid = lax.axis_index("c")
    sid = lax.axis_index("s")
    wid = sid * 2 + cid

    _zero_vmem_2d(zb_v, 16)
    for t in range(SEL_PER_TILE // 16):
        pltpu.sync_copy(zb_v, sel_sh.at[pl.ds(sid * SEL_PER_TILE + t * 16, 16)])
    plsc.subcore_barrier()

    iota16 = lax.iota(jnp.int32, 16)
    one_i = jnp.ones((16,), jnp.int32)
    zero_i = jnp.zeros((16,), jnp.int32)

    def chunk(i, carry):
        base = wid * EAPW + i * C4
        cp1 = pltpu.async_copy(fte_hbm.at[pl.ds(base, C4)], rows_v, sem1)
        pltpu.sync_copy(s2_hbm.at[pl.ds(base, C4)], s2_v)
        pltpu.sync_copy(adst_hbm.at[pl.ds(base, C4)], dst_v)
        cp1.wait()
        for g in range(C4 // 16):
            ridx = iota16 + (16 * g)
            wv = s2_v[pl.ds(16 * g, 16)]

            def scale_step(d, cidx):
                vs = plsc.load_gather(rows_v, [ridx, cidx])
                plsc.store_scatter(rows_v, [ridx, cidx], vs * wv)
                return (cidx + one_i) & 127

            lax.fori_loop(0, DIM, scale_step, iota16, unroll=8)
        pltpu.sync_copy(rows_v, sel_sh.at[dst_v], add=True)
        return carry

    lax.fori_loop(0, NCH4, chunk, 0)
    plsc.subcore_barrier()

    r0 = sid * SEL_PER_TILE
    pltpu.sync_copy(sel_sh.at[pl.ds(r0, SEL_PER_TILE)],
                    sel_hbm.at[pl.ds(cid * N_TARGET + r0, SEL_PER_TILE)])


_select_call = pl.kernel(
    _select_body,
    out_type=jax.ShapeDtypeStruct((2 * N_TARGET, DIM), jnp.float32),
    mesh=_mesh,
    compiler_params=_sc_params,
    scratch_types=[
        pltpu.VMEM((C4,), jnp.int32),
        pltpu.VMEM((C4,), jnp.float32),
        pltpu.VMEM((C4, DIM), jnp.float32),
        pltpu.VMEM((16, DIM), jnp.float32),
        pltpu.VMEM_SHARED((N_TARGET, DIM), jnp.float32),
        pltpu.SemaphoreType.DMA,
    ],
)


def _premul_body(e_ref, pw_ref, o_ref):
    o_ref[...] = e_ref[...] * pw_ref[...]


def _premul(embedding, p_w_row):
    R = 2000
    return pl.pallas_call(
        _premul_body,
        grid=(N_ITEM // R,),
        in_specs=[
            pl.BlockSpec((R, DIM), lambda i: (i, 0)),
            pl.BlockSpec((1, DIM), lambda i: (0, 0)),
        ],
        out_specs=pl.BlockSpec((R, DIM), lambda i: (i, 0)),
        out_shape=jax.ShapeDtypeStruct((N_ITEM, DIM), jnp.float32),
    )(embedding, p_w_row)


def _norm_body(f_ref, s_ref, o_ref):
    f = f_ref[0] + f_ref[1]
    s = jnp.sum(s_ref[...], axis=1) + 1e-9
    o_ref[...] = f / s[:, None]


def _normalize(ftnum2, s_parts_t):
    R = 1000
    return pl.pallas_call(
        _norm_body,
        grid=(N_ITEM // R,),
        in_specs=[
            pl.BlockSpec((2, R, DIM), lambda i: (0, i, 0)),
            pl.BlockSpec((R, s_parts_t.shape[1]), lambda i: (i, 0)),
        ],
        out_specs=pl.BlockSpec((R, DIM), lambda i: (i, 0)),
        out_shape=jax.ShapeDtypeStruct((N_ITEM, DIM), jnp.float32),
    )(ftnum2, s_parts_t)


def _s2_body(fte_ref, hp_ref, qa_ref, qb_ref, t0_ref, o_ref):
    i = pl.program_id(0)
    z = jnp.dot(fte_ref[...], qa_ref[...], preferred_element_type=jnp.float32)
    z = z + jnp.dot(hp_ref[...], qb_ref[...], preferred_element_type=jnp.float32)
    s2 = jnp.sum(jnp.tanh(z) * t0_ref[...], axis=-1).reshape(o_ref.shape)
    rid = (i * 1024
           + lax.broadcasted_iota(jnp.int32, o_ref.shape, 0) * 128
           + lax.broadcasted_iota(jnp.int32, o_ref.shape, 1))
    o_ref[...] = jnp.where(rid < E_AGG, s2, 0.0)


def _s2_compute(fte, hp, qaT, qbT, t0):
    nblk = EA_PAD // 1024
    out = pl.pallas_call(
        _s2_body,
        grid=(nblk,),
        in_specs=[
            pl.BlockSpec((1024, DIM), lambda i: (i, 0)),
            pl.BlockSpec((1024, DIM), lambda i: (i, 0)),
            pl.BlockSpec((DIM, DIM), lambda i: (0, 0)),
            pl.BlockSpec((DIM, DIM), lambda i: (0, 0)),
            pl.BlockSpec((1, DIM), lambda i: (0, 0)),
        ],
        out_specs=pl.BlockSpec((8, 128), lambda i: (i, 0)),
        out_shape=jax.ShapeDtypeStruct((nblk * 8, 128), jnp.float32),
    )(fte, hp, qaT, qbT, t0)
    return out.reshape(EA_PAD)


def _score_body(sel_ref, emb_ref, o_ref):
    sel = sel_ref[0] + sel_ref[1]
    o_ref[...] = lax.dot_general(
        sel, emb_ref[...],
        dimension_numbers=(((1,), (1,)), ((), ())),
        preferred_element_type=jnp.float32)


def _scores(sel2, emb_pad):
    NBLK = 2048
    npad = emb_pad.shape[0]
    return pl.pallas_call(
        _score_body,
        grid=(npad // NBLK,),
        in_specs=[
            pl.BlockSpec((2, N_TARGET, DIM), lambda i: (0, 0, 0)),
            pl.BlockSpec((NBLK, DIM), lambda i: (i, 0)),
        ],
        out_specs=pl.BlockSpec((N_TARGET, NBLK), lambda i: (0, i)),
        out_shape=jax.ShapeDtypeStruct((N_TARGET, npad), jnp.float32),
    )(sel2, emb_pad)


def kernel(item_ids, edge_index, pid, tid, agg_src, agg_dst,
           embedding, pos_embedding, target_embedding, p_w, q_w):
    src = edge_index[0]
    dst = edge_index[1]

    embp = _premul(embedding, p_w[None, :])
    ftnum_flat, s_parts = _edge_call(src, dst, item_ids, embedding, embp)
    ftnum2 = ftnum_flat.reshape(2, N_ITEM, DIM)
    s_parts = s_parts.reshape(NW, N_ITEM)
    ft = _normalize(ftnum2, s_parts.T)

    pad_a = EA_PAD - E_AGG
    asrc_p = jnp.pad(agg_src, (0, pad_a))
    adst_p = jnp.pad(agg_dst, (0, pad_a))
    pid_p = jnp.pad(pid, (0, pad_a))

    fte, hp = _gather_call(asrc_p, pid_p, ft, pos_embedding)

    qaT = q_w[:, :DIM].T
    qbT = q_w[:, DIM:].T
    t0 = target_embedding[0:1]
    s2 = _s2_compute(fte, hp, qaT, qbT, t0)

    sel_flat = _select_call(fte, s2, adst_p)
    sel2 = sel_flat.reshape(2, N_TARGET, DIM)

    emb_pad = jnp.pad(embedding, ((0, 10240 - NUM_NODE), (0, 0)))
    full = _scores(sel2, emb_pad)
    return full[:, 1:NUM_NODE]


# speedup vs baseline: 6.4191x; 1.0444x over previous
import functools
import jax
import jax.numpy as jnp
from jax import lax
from jax.experimental import pallas as pl
from jax.experimental.pallas import tpu as pltpu
from jax.experimental.pallas import tpu_sc as plsc

DIM = 128
ALPHA = 0.2
NUM_NODE = 10000
N_ITEM = 10000
N_TARGET = 1024
E_INT = 320000
E_AGG = 50000

NW = 32
C2 = 48
NCH2 = 209
EPW = C2 * NCH2
E_PAD = NW * EPW
NIP = 10240

EA_PAD = 50176
EAPW = EA_PAD // NW
C4 = 112
NCH4 = EAPW // C4

ROWS_PER_TILE = NIP // 16
SEL_PER_TILE = N_TARGET // 16

_mesh = plsc.VectorSubcoreMesh(core_axis_name="c", subcore_axis_name="s")
_sc_params = pltpu.CompilerParams(needs_layout_passes=False)


def _zero_vmem_2d(ref, nrows):
    z = jnp.zeros((16,), jnp.float32)
    for r in range(nrows):
        for j in range(DIM // 16):
            ref[r, pl.ds(16 * j, 16)] = z


def _edge_body(src_hbm, dst_hbm, iid_hbm, emb_hbm, embp_hbm,
               ftnum_hbm, s_hbm,
               iid_v, src_v0, dst_v0, gs_v0, gd_v0,
               src_v1, dst_v1, gs_v1, gd_v1,
               rows_s0, rows_d0, rows_s1, rows_d1,
               zb_v, s_hist, ft_sh,
               sem_s0, sem_d0, sem_s1, sem_d1):
    cid = lax.axis_index("c")
    sid = lax.axis_index("s")
    wid = sid * 2 + cid

    SRC = [src_v0, src_v1]
    DST = [dst_v0, dst_v1]
    GS = [gs_v0, gs_v1]
    GD = [gd_v0, gd_v1]
    RS = [rows_s0, rows_s1]
    RD = [rows_d0, rows_d1]
    SS = [sem_s0, sem_s1]
    SD = [sem_d0, sem_d1]

    pltpu.sync_copy(iid_hbm, iid_v)

    zf = jnp.zeros((16,), jnp.float32)
    for i in range(NIP // 16):
        s_hist[pl.ds(16 * i, 16)] = zf
    _zero_vmem_2d(zb_v, 16)
    for t in range(ROWS_PER_TILE // 16):
        pltpu.sync_copy(zb_v, ft_sh.at[pl.ds(sid * ROWS_PER_TILE + t * 16, 16)])

    plsc.subcore_barrier()

    iota16 = lax.iota(jnp.int32, 16)
    one_i = jnp.ones((16,), jnp.int32)

    def issue(c, b):
        base = wid * EPW + c * C2
        pltpu.sync_copy(src_hbm.at[pl.ds(base, C2)], SRC[b])
        pltpu.sync_copy(dst_hbm.at[pl.ds(base, C2)], DST[b])
        for j in range(C2 // 16):
            sl = pl.ds(16 * j, 16)
            GS[b][sl] = plsc.load_gather(iid_v, [SRC[b][sl]])
            GD[b][sl] = plsc.load_gather(iid_v, [DST[b][sl]])
        h1 = pltpu.async_copy(emb_hbm.at[GS[b]], RS[b], SS[b])
        h2 = pltpu.async_copy(embp_hbm.at[GD[b]], RD[b], SD[b])
        return h1, h2

    def compute(b):
        rows_s = RS[b]
        rows_d = RD[b]
        for g in range(C2 // 16):
            ridx = iota16 + (16 * g)

            def dot_step(d, c):
                a0, a1, a2, a3, c0, c1, c2, c3 = c
                v0 = plsc.load_gather(rows_s, [ridx, c0])
                w0 = plsc.load_gather(rows_d, [ridx, c0])
                v1 = plsc.load_gather(rows_s, [ridx, c1])
                w1 = plsc.load_gather(rows_d, [ridx, c1])
                v2 = plsc.load_gather(rows_s, [ridx, c2])
                w2 = plsc.load_gather(rows_d, [ridx, c2])
                v3 = plsc.load_gather(rows_s, [ridx, c3])
                w3 = plsc.load_gather(rows_d, [ridx, c3])
                return (a0 + v0 * w0, a1 + v1 * w1,
                        a2 + v2 * w2, a3 + v3 * w3,
                        (c0 + one_i) & 127, (c1 + one_i) & 127,
                        (c2 + one_i) & 127, (c3 + one_i) & 127)

            init = (zf, zf, zf, zf,
                    iota16, (iota16 + 32) & 127,
                    (iota16 + 64) & 127, (iota16 + 96) & 127)
            r = lax.fori_loop(0, DIM // 4, dot_step, init, unroll=4)
            acc = (r[0] + r[1]) + (r[2] + r[3])
            ev = jnp.maximum(acc, ALPHA * acc)
            wv = jnp.exp(ev)
            dst16 = DST[b][pl.ds(16 * g, 16)]
            plsc.addupdate_scatter(s_hist, [dst16], wv)

            def scale_step(d, c):
                c0, c1 = c
                v0 = plsc.load_gather(rows_s, [ridx, c0])
                plsc.store_scatter(rows_s, [ridx, c0], v0 * wv)
                v1 = plsc.load_gather(rows_s, [ridx, c1])
                plsc.store_scatter(rows_s, [ridx, c1], v1 * wv)
                return (c0 + one_i) & 127, (c1 + one_i) & 127

            lax.fori_loop(0, DIM // 2, scale_step,
                          (iota16, (iota16 + 64) & 127), unroll=4)

    def scatter(b):
        pltpu.sync_copy(RS[b], ft_sh.at[DST[b]], add=True)

    def pair(t, carry):
        c0 = 2 * t
        ha1, ha2 = issue(c0, 0)
        hb1, hb2 = issue(c0 + 1, 1)
        ha1.wait()
        ha2.wait()
        compute(0)
        scatter(0)
        hb1.wait()
        hb2.wait()
        compute(1)
        scatter(1)
        return carry

    lax.fori_loop(0, (NCH2 - 1) // 2, pair, 0)
    h1, h2 = issue(NCH2 - 1, 0)
    h1.wait()
    h2.wait()
    compute(0)
    scatter(0)

    plsc.subcore_barrier()

    r0 = sid * ROWS_PER_TILE
    pltpu.sync_copy(ft_sh.at[pl.ds(r0, ROWS_PER_TILE)],
                    ftnum_hbm.at[pl.ds(cid * NIP + r0, ROWS_PER_TILE)])
    pltpu.sync_copy(s_hist, s_hbm.at[pl.ds(wid * NIP, NIP)])


_edge_call = pl.kernel(
    _edge_body,
    out_type=(jax.ShapeDtypeStruct((2 * NIP, DIM), jnp.float32),
              jax.ShapeDtypeStruct((NW * NIP,), jnp.float32)),
    mesh=_mesh,
    compiler_params=_sc_params,
    scratch_types=[
        pltpu.VMEM((NIP,), jnp.int32),
        pltpu.VMEM((C2,), jnp.int32),
        pltpu.VMEM((C2,), jnp.int32),
        pltpu.VMEM((C2,), jnp.int32),
        pltpu.VMEM((C2,), jnp.int32),
        pltpu.VMEM((C2,), jnp.int32),
        pltpu.VMEM((C2,), jnp.int32),
        pltpu.VMEM((C2,), jnp.int32),
        pltpu.VMEM((C2,), jnp.int32),
        pltpu.VMEM((C2, DIM), jnp.float32),
        pltpu.VMEM((C2, DIM), jnp.float32),
        pltpu.VMEM((C2, DIM), jnp.float32),
        pltpu.VMEM((C2, DIM), jnp.float32),
        pltpu.VMEM((16, DIM), jnp.float32),
        pltpu.VMEM((NIP,), jnp.float32),
        pltpu.VMEM_SHARED((NIP, DIM), jnp.float32),
        pltpu.SemaphoreType.DMA,
        pltpu.SemaphoreType.DMA,
        pltpu.SemaphoreType.DMA,
        pltpu.SemaphoreType.DMA,
    ],
)


def _gather_body(asrc_hbm, pid_hbm, ft_hbm, pos_hbm,
                 fte_hbm, hp_hbm,
                 idx_v, idx2_v, buf1, buf2, sem1, sem2):
    cid = lax.axis_index("c")
    sid = lax.axis_index("s")
    wid = sid * 2 + cid

    def chunk(i, carry):
        base = wid * EAPW + i * C4
        pltpu.sync_copy(asrc_hbm.at[pl.ds(base, C4)], idx_v)
        pltpu.sync_copy(pid_hbm.at[pl.ds(base, C4)], idx2_v)
        cp1 = pltpu.async_copy(ft_hbm.at[idx_v], buf1, sem1)
        cp2 = pltpu.async_copy(pos_hbm.at[idx2_v], buf2, sem2)
        cp1.wait()
        cp2.wait()
        pltpu.sync_copy(buf1, fte_hbm.at[pl.ds(base, C4)])
        pltpu.sync_copy(buf2, hp_hbm.at[pl.ds(base, C4)])
        return carry

    lax.fori_loop(0, NCH4, chunk, 0)


_gather_call = pl.kernel(
    _gather_body,
    out_type=(jax.ShapeDtypeStruct((EA_PAD, DIM), jnp.float32),
              jax.ShapeDtypeStruct((EA_PAD, DIM), jnp.float32)),
    mesh=_mesh,
    compiler_params=_sc_params,
    scratch_types=[
        pltpu.VMEM((C4,), jnp.int32),
        pltpu.VMEM((C4,), jnp.int32),
        pltpu.VMEM((C4, DIM), jnp.float32),
        pltpu.VMEM((C4, DIM), jnp.float32),
        pltpu.SemaphoreType.DMA,
        pltpu.SemaphoreType.DMA,
    ],
)


def _select_body(fte_hbm, s2_hbm, adst_hbm,
                 sel_hbm,
                 dst_v, s2_v, rows_v, zb_v, sel_sh, sem1):
    cid = lax.axis_index("c")
    sid = lax.axis_index("s")
    wid = sid * 2 + cid

    _zero_vmem_2d(zb_v, 16)
    for t in range(SEL_PER_TILE // 16):
        pltpu.sync_copy(zb_v, sel_sh.at[pl.ds(sid * SEL_PER_TILE + t * 16, 16)])
    plsc.subcore_barrier()

    iota16 = lax.iota(jnp.int32, 16)
    one_i = jnp.ones((16,), jnp.int32)
    zero_i = jnp.zeros((16,), jnp.int32)

    def chunk(i, carry):
        base = wid * EAPW + i * C4
        cp1 = pltpu.async_copy(fte_hbm.at[pl.ds(base, C4)], rows_v, sem1)
        pltpu.sync_copy(s2_hbm.at[pl.ds(base, C4)], s2_v)
        pltpu.sync_copy(adst_hbm.at[pl.ds(base, C4)], dst_v)
        cp1.wait()
        for g in range(C4 // 16):
            ridx = iota16 + (16 * g)
            wv = s2_v[pl.ds(16 * g, 16)]

            def scale_step(d, cidx):
                vs = plsc.load_gather(rows_v, [ridx, cidx])
                plsc.store_scatter(rows_v, [ridx, cidx], vs * wv)
                return (cidx + one_i) & 127

            lax.fori_loop(0, DIM, scale_step, iota16, unroll=8)
        pltpu.sync_copy(rows_v, sel_sh.at[dst_v], add=True)
        return carry

    lax.fori_loop(0, NCH4, chunk, 0)
    plsc.subcore_barrier()

    r0 = sid * SEL_PER_TILE
    pltpu.sync_copy(sel_sh.at[pl.ds(r0, SEL_PER_TILE)],
                    sel_hbm.at[pl.ds(cid * N_TARGET + r0, SEL_PER_TILE)])


_select_call = pl.kernel(
    _select_body,
    out_type=jax.ShapeDtypeStruct((2 * N_TARGET, DIM), jnp.float32),
    mesh=_mesh,
    compiler_params=_sc_params,
    scratch_types=[
        pltpu.VMEM((C4,), jnp.int32),
        pltpu.VMEM((C4,), jnp.float32),
        pltpu.VMEM((C4, DIM), jnp.float32),
        pltpu.VMEM((16, DIM), jnp.float32),
        pltpu.VMEM_SHARED((N_TARGET, DIM), jnp.float32),
        pltpu.SemaphoreType.DMA,
    ],
)


def _premul_body(e_ref, pw_ref, o_ref):
    o_ref[...] = e_ref[...] * pw_ref[...]


def _premul(embedding, p_w_row):
    R = 2000
    return pl.pallas_call(
        _premul_body,
        grid=(N_ITEM // R,),
        in_specs=[
            pl.BlockSpec((R, DIM), lambda i: (i, 0)),
            pl.BlockSpec((1, DIM), lambda i: (0, 0)),
        ],
        out_specs=pl.BlockSpec((R, DIM), lambda i: (i, 0)),
        out_shape=jax.ShapeDtypeStruct((N_ITEM, DIM), jnp.float32),
    )(embedding, p_w_row)


def _norm_body(f_ref, s_ref, o_ref):
    f = f_ref[0] + f_ref[1]
    s = jnp.sum(s_ref[...], axis=1) + 1e-9
    o_ref[...] = f / s[:, None]


def _normalize(ftnum2, s_parts_t):
    R = 1000
    return pl.pallas_call(
        _norm_body,
        grid=(N_ITEM // R,),
        in_specs=[
            pl.BlockSpec((2, R, DIM), lambda i: (0, i, 0)),
            pl.BlockSpec((R, s_parts_t.shape[1]), lambda i: (i, 0)),
        ],
        out_specs=pl.BlockSpec((R, DIM), lambda i: (i, 0)),
        out_shape=jax.ShapeDtypeStruct((N_ITEM, DIM), jnp.float32),
    )(ftnum2, s_parts_t)


def _s2_body(fte_ref, hp_ref, qa_ref, qb_ref, t0_ref, o_ref):
    i = pl.program_id(0)
    z = jnp.dot(fte_ref[...], qa_ref[...], preferred_element_type=jnp.float32)
    z = z + jnp.dot(hp_ref[...], qb_ref[...], preferred_element_type=jnp.float32)
    s2 = jnp.sum(jnp.tanh(z) * t0_ref[...], axis=-1).reshape(o_ref.shape)
    rid = (i * 1024
           + lax.broadcasted_iota(jnp.int32, o_ref.shape, 0) * 128
           + lax.broadcasted_iota(jnp.int32, o_ref.shape, 1))
    o_ref[...] = jnp.where(rid < E_AGG, s2, 0.0)


def _s2_compute(fte, hp, qaT, qbT, t0):
    nblk = EA_PAD // 1024
    out = pl.pallas_call(
        _s2_body,
        grid=(nblk,),
        in_specs=[
            pl.BlockSpec((1024, DIM), lambda i: (i, 0)),
            pl.BlockSpec((1024, DIM), lambda i: (i, 0)),
            pl.BlockSpec((DIM, DIM), lambda i: (0, 0)),
            pl.BlockSpec((DIM, DIM), lambda i: (0, 0)),
            pl.BlockSpec((1, DIM), lambda i: (0, 0)),
        ],
        out_specs=pl.BlockSpec((8, 128), lambda i: (i, 0)),
        out_shape=jax.ShapeDtypeStruct((nblk * 8, 128), jnp.float32),
    )(fte, hp, qaT, qbT, t0)
    return out.reshape(EA_PAD)


def _score_body(sel_ref, emb_ref, o_ref):
    sel = sel_ref[0] + sel_ref[1]
    o_ref[...] = lax.dot_general(
        sel, emb_ref[...],
        dimension_numbers=(((1,), (1,)), ((), ())),
        preferred_element_type=jnp.float32)


def _scores(sel2, emb_pad):
    NBLK = 2048
    npad = emb_pad.shape[0]
    return pl.pallas_call(
        _score_body,
        grid=(npad // NBLK,),
        in_specs=[
            pl.BlockSpec((2, N_TARGET, DIM), lambda i: (0, 0, 0)),
            pl.BlockSpec((NBLK, DIM), lambda i: (i, 0)),
        ],
        out_specs=pl.BlockSpec((N_TARGET, NBLK), lambda i: (0, i)),
        out_shape=jax.ShapeDtypeStruct((N_TARGET, npad), jnp.float32),
    )(sel2, emb_pad)


def kernel(item_ids, edge_index, pid, tid, agg_src, agg_dst,
           embedding, pos_embedding, target_embedding, p_w, q_w):
    pad_e = E_PAD - E_INT
    src = jnp.pad(edge_index[0], (0, pad_e))
    ar = jnp.arange(E_PAD, dtype=jnp.int32)
    dst = jnp.where(ar < E_INT, jnp.pad(edge_index[1], (0, pad_e)),
                    N_ITEM + (ar & 127))
    iid_p = jnp.pad(item_ids, (0, NIP - N_ITEM))

    embp = _premul(embedding, p_w[None, :])
    ftnum_flat, s_parts = _edge_call(src, dst, iid_p, embedding, embp)
    ftnum2 = ftnum_flat.reshape(2, NIP, DIM)[:, :N_ITEM]
    s_parts = s_parts.reshape(NW, NIP)[:, :N_ITEM]
    ft = _normalize(ftnum2, s_parts.T)

    pad_a = EA_PAD - E_AGG
    asrc_p = jnp.pad(agg_src, (0, pad_a))
    adst_p = jnp.pad(agg_dst, (0, pad_a))
    pid_p = jnp.pad(pid, (0, pad_a))

    fte, hp = _gather_call(asrc_p, pid_p, ft, pos_embedding)

    qaT = q_w[:, :DIM].T
    qbT = q_w[:, DIM:].T
    t0 = target_embedding[0:1]
    s2 = _s2_compute(fte, hp, qaT, qbT, t0)

    sel_flat = _select_call(fte, s2, adst_p)
    sel2 = sel_flat.reshape(2, N_TARGET, DIM)

    emb_pad = jnp.pad(embedding, ((0, 10240 - NUM_NODE), (0, 0)))
    full = _scores(sel2, emb_pad)
    return full[:, 1:NUM_NODE]


# speedup vs baseline: 8.7196x; 1.3584x over previous
import functools
import jax
import jax.numpy as jnp
from jax import lax
from jax.experimental import pallas as pl
from jax.experimental.pallas import tpu as pltpu
from jax.experimental.pallas import tpu_sc as plsc

DIM = 128
ALPHA = 0.2
NUM_NODE = 10000
N_ITEM = 10000
N_TARGET = 1024
E_INT = 320000
E_AGG = 50000

NW = 32
C2 = 48
NCH2 = 209
EPW = C2 * NCH2
E_PAD = NW * EPW
NIP = 10240

EA_PAD = 50176
EAPW = EA_PAD // NW
C4 = 112
NCH4 = EAPW // C4

ROWS_PER_TILE = NIP // 16
SEL_PER_TILE = N_TARGET // 16

_mesh = plsc.VectorSubcoreMesh(core_axis_name="c", subcore_axis_name="s")
_sc_params = pltpu.CompilerParams(needs_layout_passes=False)


def _zero_vmem_2d(ref, nrows):
    z = jnp.zeros((16,), jnp.float32)
    for r in range(nrows):
        for j in range(DIM // 16):
            ref[r, pl.ds(16 * j, 16)] = z


def _edge_body(src_hbm, dst_hbm, iid_hbm, emb_hbm, embp_hbm,
               ftnum_hbm, s_hbm,
               iid_v, src_v0, dst_v0, gs_v0, gd_v0,
               src_v1, dst_v1, gs_v1, gd_v1,
               rows_s0, rows_d0, rows_s1, rows_d1,
               zb_v, s_hist, ft_sh,
               sem_s0, sem_d0, sem_s1, sem_d1):
    cid = lax.axis_index("c")
    sid = lax.axis_index("s")
    wid = sid * 2 + cid

    SRC = [src_v0, src_v1]
    DST = [dst_v0, dst_v1]
    GS = [gs_v0, gs_v1]
    GD = [gd_v0, gd_v1]
    RS = [rows_s0, rows_s1]
    RD = [rows_d0, rows_d1]
    SS = [sem_s0, sem_s1]
    SD = [sem_d0, sem_d1]

    pltpu.sync_copy(iid_hbm, iid_v)

    zf = jnp.zeros((16,), jnp.float32)
    for i in range(NIP // 16):
        s_hist[pl.ds(16 * i, 16)] = zf
    _zero_vmem_2d(zb_v, 16)
    for t in range(ROWS_PER_TILE // 16):
        pltpu.sync_copy(zb_v, ft_sh.at[pl.ds(sid * ROWS_PER_TILE + t * 16, 16)])

    plsc.subcore_barrier()

    iota16 = lax.iota(jnp.int32, 16)
    one_i = jnp.ones((16,), jnp.int32)

    def issue(c, b):
        base = wid * EPW + c * C2
        pltpu.sync_copy(src_hbm.at[pl.ds(base, C2)], SRC[b])
        pltpu.sync_copy(dst_hbm.at[pl.ds(base, C2)], DST[b])
        for j in range(C2 // 16):
            sl = pl.ds(16 * j, 16)
            GS[b][sl] = plsc.load_gather(iid_v, [SRC[b][sl]])
            GD[b][sl] = plsc.load_gather(iid_v, [DST[b][sl]])
        h1 = pltpu.async_copy(emb_hbm.at[GS[b]], RS[b], SS[b])
        h2 = pltpu.async_copy(embp_hbm.at[GD[b]], RD[b], SD[b])
        return h1, h2

    def compute(b):
        rows_s = RS[b]
        rows_d = RD[b]
        for g in range(C2 // 16):

            def dot_edge(kk, ev):
                k = 16 * g + kk
                acc = rows_s[k, pl.ds(0, 16)] * rows_d[k, pl.ds(0, 16)]
                for j in range(1, 8):
                    sl = pl.ds(16 * j, 16)
                    acc = acc + rows_s[k, sl] * rows_d[k, sl]
                return jnp.where(iota16 == kk, jnp.sum(acc), ev)

            ev = lax.fori_loop(0, 16, dot_edge, zf, unroll=4)
            ev = jnp.maximum(ev, ALPHA * ev)
            wv = jnp.exp(ev)
            dst16 = DST[b][pl.ds(16 * g, 16)]
            plsc.addupdate_scatter(s_hist, [dst16], wv)

            def scale_edge(kk, c):
                k = 16 * g + kk
                wk = jnp.sum(jnp.where(iota16 == kk, wv, zf))
                for j in range(8):
                    sl = pl.ds(16 * j, 16)
                    rows_s[k, sl] = rows_s[k, sl] * wk
                return c

            lax.fori_loop(0, 16, scale_edge, 0, unroll=4)

    def scatter(b):
        pltpu.sync_copy(RS[b], ft_sh.at[DST[b]], add=True)

    def pair(t, carry):
        c0 = 2 * t
        ha1, ha2 = issue(c0, 0)
        hb1, hb2 = issue(c0 + 1, 1)
        ha1.wait()
        ha2.wait()
        compute(0)
        scatter(0)
        hb1.wait()
        hb2.wait()
        compute(1)
        scatter(1)
        return carry

    lax.fori_loop(0, (NCH2 - 1) // 2, pair, 0)
    h1, h2 = issue(NCH2 - 1, 0)
    h1.wait()
    h2.wait()
    compute(0)
    scatter(0)

    plsc.subcore_barrier()

    r0 = sid * ROWS_PER_TILE
    pltpu.sync_copy(ft_sh.at[pl.ds(r0, ROWS_PER_TILE)],
                    ftnum_hbm.at[pl.ds(cid * NIP + r0, ROWS_PER_TILE)])
    pltpu.sync_copy(s_hist, s_hbm.at[pl.ds(wid * NIP, NIP)])


_edge_call = pl.kernel(
    _edge_body,
    out_type=(jax.ShapeDtypeStruct((2 * NIP, DIM), jnp.float32),
              jax.ShapeDtypeStruct((NW * NIP,), jnp.float32)),
    mesh=_mesh,
    compiler_params=_sc_params,
    scratch_types=[
        pltpu.VMEM((NIP,), jnp.int32),
        pltpu.VMEM((C2,), jnp.int32),
        pltpu.VMEM((C2,), jnp.int32),
        pltpu.VMEM((C2,), jnp.int32),
        pltpu.VMEM((C2,), jnp.int32),
        pltpu.VMEM((C2,), jnp.int32),
        pltpu.VMEM((C2,), jnp.int32),
        pltpu.VMEM((C2,), jnp.int32),
        pltpu.VMEM((C2,), jnp.int32),
        pltpu.VMEM((C2, DIM), jnp.float32),
        pltpu.VMEM((C2, DIM), jnp.float32),
        pltpu.VMEM((C2, DIM), jnp.float32),
        pltpu.VMEM((C2, DIM), jnp.float32),
        pltpu.VMEM((16, DIM), jnp.float32),
        pltpu.VMEM((NIP,), jnp.float32),
        pltpu.VMEM_SHARED((NIP, DIM), jnp.float32),
        pltpu.SemaphoreType.DMA,
        pltpu.SemaphoreType.DMA,
        pltpu.SemaphoreType.DMA,
        pltpu.SemaphoreType.DMA,
    ],
)


def _gather_body(asrc_hbm, pid_hbm, ft_hbm, pos_hbm,
                 fte_hbm, hp_hbm,
                 idx_v, idx2_v, buf1, buf2, sem1, sem2):
    cid = lax.axis_index("c")
    sid = lax.axis_index("s")
    wid = sid * 2 + cid

    def chunk(i, carry):
        base = wid * EAPW + i * C4
        pltpu.sync_copy(asrc_hbm.at[pl.ds(base, C4)], idx_v)
        pltpu.sync_copy(pid_hbm.at[pl.ds(base, C4)], idx2_v)
        cp1 = pltpu.async_copy(ft_hbm.at[idx_v], buf1, sem1)
        cp2 = pltpu.async_copy(pos_hbm.at[idx2_v], buf2, sem2)
        cp1.wait()
        cp2.wait()
        pltpu.sync_copy(buf1, fte_hbm.at[pl.ds(base, C4)])
        pltpu.sync_copy(buf2, hp_hbm.at[pl.ds(base, C4)])
        return carry

    lax.fori_loop(0, NCH4, chunk, 0)


_gather_call = pl.kernel(
    _gather_body,
    out_type=(jax.ShapeDtypeStruct((EA_PAD, DIM), jnp.float32),
              jax.ShapeDtypeStruct((EA_PAD, DIM), jnp.float32)),
    mesh=_mesh,
    compiler_params=_sc_params,
    scratch_types=[
        pltpu.VMEM((C4,), jnp.int32),
        pltpu.VMEM((C4,), jnp.int32),
        pltpu.VMEM((C4, DIM), jnp.float32),
        pltpu.VMEM((C4, DIM), jnp.float32),
        pltpu.SemaphoreType.DMA,
        pltpu.SemaphoreType.DMA,
    ],
)


def _select_body(fte_hbm, s2_hbm, adst_hbm,
                 sel_hbm,
                 dst_v, s2_v, rows_v, zb_v, sel_sh, sem1):
    cid = lax.axis_index("c")
    sid = lax.axis_index("s")
    wid = sid * 2 + cid

    _zero_vmem_2d(zb_v, 16)
    for t in range(SEL_PER_TILE // 16):
        pltpu.sync_copy(zb_v, sel_sh.at[pl.ds(sid * SEL_PER_TILE + t * 16, 16)])
    plsc.subcore_barrier()

    iota16 = lax.iota(jnp.int32, 16)
    one_i = jnp.ones((16,), jnp.int32)
    zero_i = jnp.zeros((16,), jnp.int32)

    def chunk(i, carry):
        base = wid * EAPW + i * C4
        cp1 = pltpu.async_copy(fte_hbm.at[pl.ds(base, C4)], rows_v, sem1)
        pltpu.sync_copy(s2_hbm.at[pl.ds(base, C4)], s2_v)
        pltpu.sync_copy(adst_hbm.at[pl.ds(base, C4)], dst_v)
        cp1.wait()
        for g in range(C4 // 16):
            ridx = iota16 + (16 * g)
            wv = s2_v[pl.ds(16 * g, 16)]

            def scale_step(d, cidx):
                vs = plsc.load_gather(rows_v, [ridx, cidx])
                plsc.store_scatter(rows_v, [ridx, cidx], vs * wv)
                return (cidx + one_i) & 127

            lax.fori_loop(0, DIM, scale_step, iota16, unroll=8)
        pltpu.sync_copy(rows_v, sel_sh.at[dst_v], add=True)
        return carry

    lax.fori_loop(0, NCH4, chunk, 0)
    plsc.subcore_barrier()

    r0 = sid * SEL_PER_TILE
    pltpu.sync_copy(sel_sh.at[pl.ds(r0, SEL_PER_TILE)],
                    sel_hbm.at[pl.ds(cid * N_TARGET + r0, SEL_PER_TILE)])


_select_call = pl.kernel(
    _select_body,
    out_type=jax.ShapeDtypeStruct((2 * N_TARGET, DIM), jnp.float32),
    mesh=_mesh,
    compiler_params=_sc_params,
    scratch_types=[
        pltpu.VMEM((C4,), jnp.int32),
        pltpu.VMEM((C4,), jnp.float32),
        pltpu.VMEM((C4, DIM), jnp.float32),
        pltpu.VMEM((16, DIM), jnp.float32),
        pltpu.VMEM_SHARED((N_TARGET, DIM), jnp.float32),
        pltpu.SemaphoreType.DMA,
    ],
)


def _premul_body(e_ref, pw_ref, o_ref):
    o_ref[...] = e_ref[...] * pw_ref[...]


def _premul(embedding, p_w_row):
    R = 2000
    return pl.pallas_call(
        _premul_body,
        grid=(N_ITEM // R,),
        in_specs=[
            pl.BlockSpec((R, DIM), lambda i: (i, 0)),
            pl.BlockSpec((1, DIM), lambda i: (0, 0)),
        ],
        out_specs=pl.BlockSpec((R, DIM), lambda i: (i, 0)),
        out_shape=jax.ShapeDtypeStruct((N_ITEM, DIM), jnp.float32),
    )(embedding, p_w_row)


def _norm_body(f_ref, s_ref, o_ref):
    f = f_ref[0] + f_ref[1]
    s = jnp.sum(s_ref[...], axis=1) + 1e-9
    o_ref[...] = f / s[:, None]


def _normalize(ftnum2, s_parts_t):
    R = 1000
    return pl.pallas_call(
        _norm_body,
        grid=(N_ITEM // R,),
        in_specs=[
            pl.BlockSpec((2, R, DIM), lambda i: (0, i, 0)),
            pl.BlockSpec((R, s_parts_t.shape[1]), lambda i: (i, 0)),
        ],
        out_specs=pl.BlockSpec((R, DIM), lambda i: (i, 0)),
        out_shape=jax.ShapeDtypeStruct((N_ITEM, DIM), jnp.float32),
    )(ftnum2, s_parts_t)


def _s2_body(fte_ref, hp_ref, qa_ref, qb_ref, t0_ref, o_ref):
    i = pl.program_id(0)
    z = jnp.dot(fte_ref[...], qa_ref[...], preferred_element_type=jnp.float32)
    z = z + jnp.dot(hp_ref[...], qb_ref[...], preferred_element_type=jnp.float32)
    s2 = jnp.sum(jnp.tanh(z) * t0_ref[...], axis=-1).reshape(o_ref.shape)
    rid = (i * 1024
           + lax.broadcasted_iota(jnp.int32, o_ref.shape, 0) * 128
           + lax.broadcasted_iota(jnp.int32, o_ref.shape, 1))
    o_ref[...] = jnp.where(rid < E_AGG, s2, 0.0)


def _s2_compute(fte, hp, qaT, qbT, t0):
    nblk = EA_PAD // 1024
    out = pl.pallas_call(
        _s2_body,
        grid=(nblk,),
        in_specs=[
            pl.BlockSpec((1024, DIM), lambda i: (i, 0)),
            pl.BlockSpec((1024, DIM), lambda i: (i, 0)),
            pl.BlockSpec((DIM, DIM), lambda i: (0, 0)),
            pl.BlockSpec((DIM, DIM), lambda i: (0, 0)),
            pl.BlockSpec((1, DIM), lambda i: (0, 0)),
        ],
        out_specs=pl.BlockSpec((8, 128), lambda i: (i, 0)),
        out_shape=jax.ShapeDtypeStruct((nblk * 8, 128), jnp.float32),
    )(fte, hp, qaT, qbT, t0)
    return out.reshape(EA_PAD)


def _score_body(sel_ref, emb_ref, o_ref):
    sel = sel_ref[0] + sel_ref[1]
    o_ref[...] = lax.dot_general(
        sel, emb_ref[...],
        dimension_numbers=(((1,), (1,)), ((), ())),
        preferred_element_type=jnp.float32)


def _scores(sel2, emb_pad):
    NBLK = 2048
    npad = emb_pad.shape[0]
    return pl.pallas_call(
        _score_body,
        grid=(npad // NBLK,),
        in_specs=[
            pl.BlockSpec((2, N_TARGET, DIM), lambda i: (0, 0, 0)),
            pl.BlockSpec((NBLK, DIM), lambda i: (i, 0)),
        ],
        out_specs=pl.BlockSpec((N_TARGET, NBLK), lambda i: (0, i)),
        out_shape=jax.ShapeDtypeStruct((N_TARGET, npad), jnp.float32),
    )(sel2, emb_pad)


def kernel(item_ids, edge_index, pid, tid, agg_src, agg_dst,
           embedding, pos_embedding, target_embedding, p_w, q_w):
    pad_e = E_PAD - E_INT
    src = jnp.pad(edge_index[0], (0, pad_e))
    ar = jnp.arange(E_PAD, dtype=jnp.int32)
    dst = jnp.where(ar < E_INT, jnp.pad(edge_index[1], (0, pad_e)),
                    N_ITEM + (ar & 127))
    iid_p = jnp.pad(item_ids, (0, NIP - N_ITEM))

    embp = _premul(embedding, p_w[None, :])
    ftnum_flat, s_parts = _edge_call(src, dst, iid_p, embedding, embp)
    ftnum2 = ftnum_flat.reshape(2, NIP, DIM)[:, :N_ITEM]
    s_parts = s_parts.reshape(NW, NIP)[:, :N_ITEM]
    ft = _normalize(ftnum2, s_parts.T)

    pad_a = EA_PAD - E_AGG
    asrc_p = jnp.pad(agg_src, (0, pad_a))
    adst_p = jnp.pad(agg_dst, (0, pad_a))
    pid_p = jnp.pad(pid, (0, pad_a))

    fte, hp = _gather_call(asrc_p, pid_p, ft, pos_embedding)

    qaT = q_w[:, :DIM].T
    qbT = q_w[:, DIM:].T
    t0 = target_embedding[0:1]
    s2 = _s2_compute(fte, hp, qaT, qbT, t0)

    sel_flat = _select_call(fte, s2, adst_p)
    sel2 = sel_flat.reshape(2, N_TARGET, DIM)

    emb_pad = jnp.pad(embedding, ((0, 10240 - NUM_NODE), (0, 0)))
    full = _scores(sel2, emb_pad)
    return full[:, 1:NUM_NODE]


# speedup vs baseline: 9.3371x; 1.0708x over previous
import functools
import jax
import jax.numpy as jnp
from jax import lax
from jax.experimental import pallas as pl
from jax.experimental.pallas import tpu as pltpu
from jax.experimental.pallas import tpu_sc as plsc

DIM = 128
ALPHA = 0.2
NUM_NODE = 10000
N_ITEM = 10000
N_TARGET = 1024
E_INT = 320000
E_AGG = 50000

NW = 32
C2 = 48
NCH2 = 209
EPW = C2 * NCH2
E_PAD = NW * EPW
NIP = 10240

EA_PAD = 50176
EAPW = EA_PAD // NW
C4 = 112
NCH4 = EAPW // C4

ROWS_PER_TILE = NIP // 16
SEL_PER_TILE = N_TARGET // 16

_mesh = plsc.VectorSubcoreMesh(core_axis_name="c", subcore_axis_name="s")
_sc_params = pltpu.CompilerParams(needs_layout_passes=False)


def _zero_vmem_2d(ref, nrows):
    z = jnp.zeros((16,), jnp.float32)
    for r in range(nrows):
        for j in range(DIM // 16):
            ref[r, pl.ds(16 * j, 16)] = z


def _edge_body(src_hbm, dst_hbm, iid_hbm, emb_hbm, embp_hbm,
               ftnum_hbm, s_hbm,
               iid_v, src_v0, dst_v0, gs_v0, gd_v0,
               src_v1, dst_v1, gs_v1, gd_v1,
               rows_s0, rows_d0, rows_s1, rows_d1,
               zb_v, s_hist, ft_sh,
               sem_s0, sem_d0, sem_s1, sem_d1):
    cid = lax.axis_index("c")
    sid = lax.axis_index("s")
    wid = sid * 2 + cid

    SRC = [src_v0, src_v1]
    DST = [dst_v0, dst_v1]
    GS = [gs_v0, gs_v1]
    GD = [gd_v0, gd_v1]
    RS = [rows_s0, rows_s1]
    RD = [rows_d0, rows_d1]
    SS = [sem_s0, sem_s1]
    SD = [sem_d0, sem_d1]

    pltpu.sync_copy(iid_hbm, iid_v)

    zf = jnp.zeros((16,), jnp.float32)
    for i in range(NIP // 16):
        s_hist[pl.ds(16 * i, 16)] = zf
    _zero_vmem_2d(zb_v, 16)
    for t in range(ROWS_PER_TILE // 16):
        pltpu.sync_copy(zb_v, ft_sh.at[pl.ds(sid * ROWS_PER_TILE + t * 16, 16)])

    plsc.subcore_barrier()

    iota16 = lax.iota(jnp.int32, 16)
    one_i = jnp.ones((16,), jnp.int32)

    def issue(c, b):
        base = wid * EPW + c * C2
        pltpu.sync_copy(src_hbm.at[pl.ds(base, C2)], SRC[b])
        pltpu.sync_copy(dst_hbm.at[pl.ds(base, C2)], DST[b])
        for j in range(C2 // 16):
            sl = pl.ds(16 * j, 16)
            GS[b][sl] = plsc.load_gather(iid_v, [SRC[b][sl]])
            GD[b][sl] = plsc.load_gather(iid_v, [DST[b][sl]])
        h1 = pltpu.async_copy(emb_hbm.at[GS[b]], RS[b], SS[b])
        h2 = pltpu.async_copy(embp_hbm.at[GD[b]], RD[b], SD[b])
        return h1, h2

    def compute(b):
        rows_s = RS[b]
        rows_d = RD[b]
        for g in range(C2 // 16):

            def dot_edge(kk, ev):
                k = 16 * g + kk
                acc = rows_s[k, pl.ds(0, 16)] * rows_d[k, pl.ds(0, 16)]
                for j in range(1, 8):
                    sl = pl.ds(16 * j, 16)
                    acc = acc + rows_s[k, sl] * rows_d[k, sl]
                return jnp.where(iota16 == kk, jnp.sum(acc), ev)

            ev = lax.fori_loop(0, 16, dot_edge, zf, unroll=4)
            ev = jnp.maximum(ev, ALPHA * ev)
            wv = jnp.exp(ev)
            dst16 = DST[b][pl.ds(16 * g, 16)]
            plsc.addupdate_scatter(s_hist, [dst16], wv)

            def scale_edge(kk, c):
                k = 16 * g + kk
                wk = jnp.sum(jnp.where(iota16 == kk, wv, zf))
                for j in range(8):
                    sl = pl.ds(16 * j, 16)
                    rows_s[k, sl] = rows_s[k, sl] * wk
                return c

            lax.fori_loop(0, 16, scale_edge, 0, unroll=4)

    def scatter(b):
        pltpu.sync_copy(RS[b], ft_sh.at[DST[b]], add=True)

    def pair(t, carry):
        c0 = 2 * t
        ha1, ha2 = issue(c0, 0)
        hb1, hb2 = issue(c0 + 1, 1)
        ha1.wait()
        ha2.wait()
        compute(0)
        scatter(0)
        hb1.wait()
        hb2.wait()
        compute(1)
        scatter(1)
        return carry

    lax.fori_loop(0, (NCH2 - 1) // 2, pair, 0)
    h1, h2 = issue(NCH2 - 1, 0)
    h1.wait()
    h2.wait()
    compute(0)
    scatter(0)

    plsc.subcore_barrier()

    r0 = sid * ROWS_PER_TILE
    pltpu.sync_copy(ft_sh.at[pl.ds(r0, ROWS_PER_TILE)],
                    ftnum_hbm.at[pl.ds(cid * NIP + r0, ROWS_PER_TILE)])
    pltpu.sync_copy(s_hist, s_hbm.at[pl.ds(wid * NIP, NIP)])


_edge_call = pl.kernel(
    _edge_body,
    out_type=(jax.ShapeDtypeStruct((2 * NIP, DIM), jnp.float32),
              jax.ShapeDtypeStruct((NW * NIP,), jnp.float32)),
    mesh=_mesh,
    compiler_params=_sc_params,
    scratch_types=[
        pltpu.VMEM((NIP,), jnp.int32),
        pltpu.VMEM((C2,), jnp.int32),
        pltpu.VMEM((C2,), jnp.int32),
        pltpu.VMEM((C2,), jnp.int32),
        pltpu.VMEM((C2,), jnp.int32),
        pltpu.VMEM((C2,), jnp.int32),
        pltpu.VMEM((C2,), jnp.int32),
        pltpu.VMEM((C2,), jnp.int32),
        pltpu.VMEM((C2,), jnp.int32),
        pltpu.VMEM((C2, DIM), jnp.float32),
        pltpu.VMEM((C2, DIM), jnp.float32),
        pltpu.VMEM((C2, DIM), jnp.float32),
        pltpu.VMEM((C2, DIM), jnp.float32),
        pltpu.VMEM((16, DIM), jnp.float32),
        pltpu.VMEM((NIP,), jnp.float32),
        pltpu.VMEM_SHARED((NIP, DIM), jnp.float32),
        pltpu.SemaphoreType.DMA,
        pltpu.SemaphoreType.DMA,
        pltpu.SemaphoreType.DMA,
        pltpu.SemaphoreType.DMA,
    ],
)


def _gather_body(asrc_hbm, pid_hbm, ft_hbm, pos_hbm,
                 fte_hbm, hp_hbm,
                 idx_v, idx2_v, buf1, buf2, sem1, sem2):
    cid = lax.axis_index("c")
    sid = lax.axis_index("s")
    wid = sid * 2 + cid

    def chunk(i, carry):
        base = wid * EAPW + i * C4
        pltpu.sync_copy(asrc_hbm.at[pl.ds(base, C4)], idx_v)
        pltpu.sync_copy(pid_hbm.at[pl.ds(base, C4)], idx2_v)
        cp1 = pltpu.async_copy(ft_hbm.at[idx_v], buf1, sem1)
        cp2 = pltpu.async_copy(pos_hbm.at[idx2_v], buf2, sem2)
        cp1.wait()
        cp2.wait()
        pltpu.sync_copy(buf1, fte_hbm.at[pl.ds(base, C4)])
        pltpu.sync_copy(buf2, hp_hbm.at[pl.ds(base, C4)])
        return carry

    lax.fori_loop(0, NCH4, chunk, 0)


_gather_call = pl.kernel(
    _gather_body,
    out_type=(jax.ShapeDtypeStruct((EA_PAD, DIM), jnp.float32),
              jax.ShapeDtypeStruct((EA_PAD, DIM), jnp.float32)),
    mesh=_mesh,
    compiler_params=_sc_params,
    scratch_types=[
        pltpu.VMEM((C4,), jnp.int32),
        pltpu.VMEM((C4,), jnp.int32),
        pltpu.VMEM((C4, DIM), jnp.float32),
        pltpu.VMEM((C4, DIM), jnp.float32),
        pltpu.SemaphoreType.DMA,
        pltpu.SemaphoreType.DMA,
    ],
)


def _select_body(fte_hbm, s2_hbm, adst_hbm,
                 sel_hbm,
                 dst_v, s2_v, rows_v, zb_v, sel_sh, sem1):
    cid = lax.axis_index("c")
    sid = lax.axis_index("s")
    wid = sid * 2 + cid

    zf = jnp.zeros((16,), jnp.float32)
    _zero_vmem_2d(zb_v, 16)
    for t in range(SEL_PER_TILE // 16):
        pltpu.sync_copy(zb_v, sel_sh.at[pl.ds(sid * SEL_PER_TILE + t * 16, 16)])
    plsc.subcore_barrier()

    iota16 = lax.iota(jnp.int32, 16)
    one_i = jnp.ones((16,), jnp.int32)
    zero_i = jnp.zeros((16,), jnp.int32)

    def chunk(i, carry):
        base = wid * EAPW + i * C4
        cp1 = pltpu.async_copy(fte_hbm.at[pl.ds(base, C4)], rows_v, sem1)
        pltpu.sync_copy(s2_hbm.at[pl.ds(base, C4)], s2_v)
        pltpu.sync_copy(adst_hbm.at[pl.ds(base, C4)], dst_v)
        cp1.wait()
        for g in range(C4 // 16):
            wv = s2_v[pl.ds(16 * g, 16)]

            def scale_edge(kk, c):
                k = 16 * g + kk
                wk = jnp.sum(jnp.where(iota16 == kk, wv, zf))
                for j in range(8):
                    sl = pl.ds(16 * j, 16)
                    rows_v[k, sl] = rows_v[k, sl] * wk
                return c

            lax.fori_loop(0, 16, scale_edge, 0, unroll=4)
        pltpu.sync_copy(rows_v, sel_sh.at[dst_v], add=True)
        return carry

    lax.fori_loop(0, NCH4, chunk, 0)
    plsc.subcore_barrier()

    r0 = sid * SEL_PER_TILE
    pltpu.sync_copy(sel_sh.at[pl.ds(r0, SEL_PER_TILE)],
                    sel_hbm.at[pl.ds(cid * N_TARGET + r0, SEL_PER_TILE)])


_select_call = pl.kernel(
    _select_body,
    out_type=jax.ShapeDtypeStruct((2 * N_TARGET, DIM), jnp.float32),
    mesh=_mesh,
    compiler_params=_sc_params,
    scratch_types=[
        pltpu.VMEM((C4,), jnp.int32),
        pltpu.VMEM((C4,), jnp.float32),
        pltpu.VMEM((C4, DIM), jnp.float32),
        pltpu.VMEM((16, DIM), jnp.float32),
        pltpu.VMEM_SHARED((N_TARGET, DIM), jnp.float32),
        pltpu.SemaphoreType.DMA,
    ],
)


def _premul_body(e_ref, pw_ref, o_ref):
    o_ref[...] = e_ref[...] * pw_ref[...]


def _premul(embedding, p_w_row):
    R = 2000
    return pl.pallas_call(
        _premul_body,
        grid=(N_ITEM // R,),
        in_specs=[
            pl.BlockSpec((R, DIM), lambda i: (i, 0)),
            pl.BlockSpec((1, DIM), lambda i: (0, 0)),
        ],
        out_specs=pl.BlockSpec((R, DIM), lambda i: (i, 0)),
        out_shape=jax.ShapeDtypeStruct((N_ITEM, DIM), jnp.float32),
    )(embedding, p_w_row)


def _norm_body(f_ref, s_ref, o_ref):
    f = f_ref[0] + f_ref[1]
    s = jnp.sum(s_ref[...], axis=1) + 1e-9
    o_ref[...] = f / s[:, None]


def _normalize(ftnum2, s_parts_t):
    R = 1000
    return pl.pallas_call(
        _norm_body,
        grid=(N_ITEM // R,),
        in_specs=[
            pl.BlockSpec((2, R, DIM), lambda i: (0, i, 0)),
            pl.BlockSpec((R, s_parts_t.shape[1]), lambda i: (i, 0)),
        ],
        out_specs=pl.BlockSpec((R, DIM), lambda i: (i, 0)),
        out_shape=jax.ShapeDtypeStruct((N_ITEM, DIM), jnp.float32),
    )(ftnum2, s_parts_t)


def _s2_body(fte_ref, hp_ref, qa_ref, qb_ref, t0_ref, o_ref):
    i = pl.program_id(0)
    z = jnp.dot(fte_ref[...], qa_ref[...], preferred_element_type=jnp.float32)
    z = z + jnp.dot(hp_ref[...], qb_ref[...], preferred_element_type=jnp.float32)
    s2 = jnp.sum(jnp.tanh(z) * t0_ref[...], axis=-1).reshape(o_ref.shape)
    rid = (i * 1024
           + lax.broadcasted_iota(jnp.int32, o_ref.shape, 0) * 128
           + lax.broadcasted_iota(jnp.int32, o_ref.shape, 1))
    o_ref[...] = jnp.where(rid < E_AGG, s2, 0.0)


def _s2_compute(fte, hp, qaT, qbT, t0):
    nblk = EA_PAD // 1024
    out = pl.pallas_call(
        _s2_body,
        grid=(nblk,),
        in_specs=[
            pl.BlockSpec((1024, DIM), lambda i: (i, 0)),
            pl.BlockSpec((1024, DIM), lambda i: (i, 0)),
            pl.BlockSpec((DIM, DIM), lambda i: (0, 0)),
            pl.BlockSpec((DIM, DIM), lambda i: (0, 0)),
            pl.BlockSpec((1, DIM), lambda i: (0, 0)),
        ],
        out_specs=pl.BlockSpec((8, 128), lambda i: (i, 0)),
        out_shape=jax.ShapeDtypeStruct((nblk * 8, 128), jnp.float32),
    )(fte, hp, qaT, qbT, t0)
    return out.reshape(EA_PAD)


def _score_body(sel_ref, emb_ref, o_ref):
    sel = sel_ref[0] + sel_ref[1]
    o_ref[...] = lax.dot_general(
        sel, emb_ref[...],
        dimension_numbers=(((1,), (1,)), ((), ())),
        preferred_element_type=jnp.float32)


def _scores(sel2, emb_pad):
    NBLK = 2048
    npad = emb_pad.shape[0]
    return pl.pallas_call(
        _score_body,
        grid=(npad // NBLK,),
        in_specs=[
            pl.BlockSpec((2, N_TARGET, DIM), lambda i: (0, 0, 0)),
            pl.BlockSpec((NBLK, DIM), lambda i: (i, 0)),
        ],
        out_specs=pl.BlockSpec((N_TARGET, NBLK), lambda i: (0, i)),
        out_shape=jax.ShapeDtypeStruct((N_TARGET, npad), jnp.float32),
    )(sel2, emb_pad)


def kernel(item_ids, edge_index, pid, tid, agg_src, agg_dst,
           embedding, pos_embedding, target_embedding, p_w, q_w):
    pad_e = E_PAD - E_INT
    src = jnp.pad(edge_index[0], (0, pad_e))
    ar = jnp.arange(E_PAD, dtype=jnp.int32)
    dst = jnp.where(ar < E_INT, jnp.pad(edge_index[1], (0, pad_e)),
                    N_ITEM + (ar & 127))
    iid_p = jnp.pad(item_ids, (0, NIP - N_ITEM))

    embp = _premul(embedding, p_w[None, :])
    ftnum_flat, s_parts = _edge_call(src, dst, iid_p, embedding, embp)
    ftnum2 = ftnum_flat.reshape(2, NIP, DIM)[:, :N_ITEM]
    s_parts = s_parts.reshape(NW, NIP)[:, :N_ITEM]
    ft = _normalize(ftnum2, s_parts.T)

    pad_a = EA_PAD - E_AGG
    asrc_p = jnp.pad(agg_src, (0, pad_a))
    adst_p = jnp.pad(agg_dst, (0, pad_a))
    pid_p = jnp.pad(pid, (0, pad_a))

    fte, hp = _gather_call(asrc_p, pid_p, ft, pos_embedding)

    qaT = q_w[:, :DIM].T
    qbT = q_w[:, DIM:].T
    t0 = target_embedding[0:1]
    s2 = _s2_compute(fte, hp, qaT, qbT, t0)

    sel_flat = _select_call(fte, s2, adst_p)
    sel2 = sel_flat.reshape(2, N_TARGET, DIM)

    emb_pad = jnp.pad(embedding, ((0, 10240 - NUM_NODE), (0, 0)))
    full = _scores(sel2, emb_pad)
    return full[:, 1:NUM_NODE]
